# baseline (speedup 1.0000x reference)
import functools

import jax
import jax.numpy as jnp
from jax import lax
from jax.experimental import pallas as pl
from jax.experimental.pallas import tpu as pltpu
from jax.experimental.pallas import tpu_sc as plsc

N_CORES = 2
N_SUBCORES = 16
LANES = 16
NW = N_CORES * N_SUBCORES
K_EDGES = 128
CORE0_FRAC = 0.74


def _lane_broadcast(v, lane):
    idx = jnp.full((LANES,), lane, jnp.int32)
    return lax.gather(
        v, idx[:, None],
        lax.GatherDimensionNumbers(
            offset_dims=(), collapsed_slice_dims=(0,), start_index_map=(0,)),
        (1,), mode=lax.GatherScatterMode.PROMISE_IN_BOUNDS)


def _split_chunks(n_edges):
    total = -(-n_edges // K_EDGES)
    cpt0 = -(-int(total * CORE0_FRAC) // N_SUBCORES)
    cpt0 = cpt0 + (cpt0 % 2)
    rest = max(total - N_SUBCORES * cpt0, 0)
    cpt1 = -(-rest // N_SUBCORES)
    cpt1 = max(cpt1 + (cpt1 % 2), 2)
    return cpt0, cpt1


def _sc_aggregate(x, idx_r, cpt0, cpt1, n_nodes):
    feat = x.shape[1]
    n_fg = feat // LANES
    n_pad = -(-n_nodes // (N_SUBCORES * K_EDGES)) * N_SUBCORES * K_EDGES
    rows_per_tile = n_pad // N_SUBCORES

    @functools.partial(
        pl.kernel,
        out_type=jax.ShapeDtypeStruct((N_CORES, n_pad, feat), jnp.float32),
        mesh=plsc.VectorSubcoreMesh(core_axis_name="c", subcore_axis_name="s"),
        scratch_types=[
            pltpu.VMEM((3, K_EDGES), jnp.int32),
            pltpu.VMEM((3, K_EDGES), jnp.int32),
            pltpu.VMEM((K_EDGES, feat), jnp.float32),
            pltpu.VMEM((K_EDGES, feat), jnp.float32),
            pltpu.VMEM_SHARED((n_pad, feat), jnp.float32),
            pltpu.SemaphoreType.DMA,
            pltpu.SemaphoreType.DMA,
            pltpu.SemaphoreType.DMA,
            pltpu.SemaphoreType.DMA,
        ],
    )
    def sc_kernel(x_hbm, idx_hbm, out_hbm,
                  idx0, idx1, rows0, rows1, agg,
                  isem0, isem1, gsem0, gsem1):
        c = lax.axis_index("c")
        s = lax.axis_index("s")
        wid = c * N_SUBCORES + s
        cpt = jnp.where(c == 0, cpt0, cpt1)
        idx_v = (idx0, idx1)
        rows_v = (rows0, rows1)
        isem = (isem0, isem1)
        gsem = (gsem0, gsem1)

        def zero_row(r, carry):
            for j in range(n_fg):
                rows0[r, pl.ds(j * LANES, LANES)] = jnp.zeros(
                    (LANES,), jnp.float32)
            return carry
        lax.fori_loop(0, K_EDGES, zero_row, 0)

        base = s * rows_per_tile
        n_full = rows_per_tile // K_EDGES
        for t in range(n_full):
            pltpu.sync_copy(rows0, agg.at[pl.ds(base + t * K_EDGES, K_EDGES)])
        plsc.subcore_barrier()

        pltpu.sync_copy(idx_hbm.at[wid, 0], idx0)
        pltpu.async_copy(x_hbm.at[idx0.at[0]], rows0, gsem0)
        pltpu.async_copy(idx_hbm.at[wid, 1], idx1, isem1)

        def pair_body(k, carry):
            for u in range(2):
                i = 2 * k + u
                slot = u
                nslot = 1 - u
                pltpu.make_async_copy(
                    x_hbm.at[idx_v[slot].at[0]], rows_v[slot],
                    gsem[slot]).wait()

                @pl.when(i + 1 < cpt)
                def _():
                    pltpu.make_async_copy(
                        idx_hbm.at[wid, i + 1], idx_v[nslot],
                        isem[nslot]).wait()
                    pltpu.async_copy(
                        x_hbm.at[idx_v[nslot].at[0]], rows_v[nslot],
                        gsem[nslot])

                def group_body(g, carry2):
                    wv = lax.bitcast_convert_type(
                        idx_v[slot][2, pl.ds(g * LANES, LANES)], jnp.float32)
                    for l in range(LANES):
                        wb = _lane_broadcast(wv, l)
                        for j in range(n_fg):
                            sl = pl.ds(j * LANES, LANES)
                            rows_v[slot][g * LANES + l, sl] = \
                                rows_v[slot][g * LANES + l, sl] * wb
                    return carry2
                lax.fori_loop(0, K_EDGES // LANES, group_body, 0)

                pltpu.sync_copy(rows_v[slot], agg.at[idx_v[slot].at[1]],
                                add=True)

                @pl.when(i + 2 < cpt)
                def _():
                    pltpu.async_copy(
                        idx_hbm.at[wid, i + 2], idx_v[slot], isem[slot])
            return carry
        lax.fori_loop(0, cpt // 2, pair_body, 0)

        plsc.subcore_barrier()
        pltpu.sync_copy(agg.at[pl.ds(base, rows_per_tile)],
                        out_hbm.at[c, pl.ds(base, rows_per_tile)])

    return sc_kernel(x, idx_r)


def _project(parts, W, b, m):
    feat = parts.shape[2]
    bm = 1000

    def mm_kernel(p_ref, w_ref, b_ref, o_ref):
        acc = p_ref[0] + p_ref[1]
        o_ref[...] = jnp.dot(acc, w_ref[...],
                             preferred_element_type=jnp.float32) \
            + b_ref[...][None, :]

    return pl.pallas_call(
        mm_kernel,
        grid=(m // bm,),
        in_specs=[
            pl.BlockSpec((N_CORES, bm, feat), lambda i: (0, i, 0)),
            pl.BlockSpec((feat, feat), lambda i: (0, 0)),
            pl.BlockSpec((feat,), lambda i: (0,)),
        ],
        out_specs=pl.BlockSpec((bm, feat), lambda i: (i, 0)),
        out_shape=jax.ShapeDtypeStruct((m, feat), jnp.float32),
    )(parts, W, b)


def kernel(x, edge_index, edge_weight, W, b):
    n_nodes = x.shape[0]
    n_edges = edge_weight.shape[0]
    cpt0, cpt1 = _split_chunks(n_edges)
    cpt_max = max(cpt0, cpt1)
    padded = N_SUBCORES * (cpt0 + cpt1) * K_EDGES
    pad = padded - n_edges

    dst = edge_index[0]
    src = edge_index[1]
    ew = edge_weight
    if pad:
        dst = jnp.concatenate([dst, jnp.zeros((pad,), dst.dtype)])
        src = jnp.concatenate([src, jnp.zeros((pad,), src.dtype)])
        ew = jnp.concatenate([ew, jnp.zeros((pad,), ew.dtype)])

    def to_tiles(a):
        n0 = N_SUBCORES * cpt0 * K_EDGES
        a0 = a[:n0].reshape(N_SUBCORES, cpt0, K_EDGES)
        a1 = a[n0:].reshape(N_SUBCORES, cpt1, K_EDGES)
        a0 = jnp.pad(a0, ((0, 0), (0, cpt_max - cpt0), (0, 0)))
        a1 = jnp.pad(a1, ((0, 0), (0, cpt_max - cpt1), (0, 0)))
        return jnp.concatenate([a0, a1], axis=0)

    idx_r = jnp.stack(
        [to_tiles(src),
         to_tiles(dst),
         to_tiles(lax.bitcast_convert_type(ew, jnp.int32))],
        axis=2)

    parts = _sc_aggregate(x, idx_r, cpt0, cpt1, n_nodes)
    return _project(parts, W, b, n_nodes)


# speedup vs baseline: 1.0898x; 1.0898x over previous
import functools

import jax
import jax.numpy as jnp
from jax import lax
from jax.experimental import pallas as pl
from jax.experimental.pallas import tpu as pltpu
from jax.experimental.pallas import tpu_sc as plsc

N_CORES = 2
N_SUBCORES = 16
LANES = 16
NW = N_CORES * N_SUBCORES
K_EDGES = 128
CORE0_FRAC = 0.76


def _lane_broadcast(v, lane):
    idx = jnp.full((LANES,), lane, jnp.int32)
    return lax.gather(
        v, idx[:, None],
        lax.GatherDimensionNumbers(
            offset_dims=(), collapsed_slice_dims=(0,), start_index_map=(0,)),
        (1,), mode=lax.GatherScatterMode.PROMISE_IN_BOUNDS)


def _split_chunks(n_edges):
    total = -(-n_edges // K_EDGES)
    cpt0 = -(-int(total * CORE0_FRAC) // N_SUBCORES)
    cpt0 = cpt0 + (cpt0 % 2)
    rest = max(total - N_SUBCORES * cpt0, 0)
    cpt1 = -(-rest // N_SUBCORES)
    cpt1 = max(cpt1 + (cpt1 % 2), 2)
    return cpt0, cpt1


def _sc_aggregate(x, idx_r, cpt0, cpt1, n_nodes):
    feat = x.shape[1]
    n_fg = feat // LANES
    n_pad = -(-n_nodes // (N_SUBCORES * K_EDGES)) * N_SUBCORES * K_EDGES
    rows_per_tile = n_pad // N_SUBCORES

    @functools.partial(
        pl.kernel,
        out_type=jax.ShapeDtypeStruct((N_CORES, n_pad, feat), jnp.float32),
        mesh=plsc.VectorSubcoreMesh(core_axis_name="c", subcore_axis_name="s"),
        scratch_types=[
            pltpu.VMEM((3, K_EDGES), jnp.int32),
            pltpu.VMEM((3, K_EDGES), jnp.int32),
            pltpu.VMEM((K_EDGES, feat), jnp.float32),
            pltpu.VMEM((K_EDGES, feat), jnp.float32),
            pltpu.VMEM_SHARED((n_pad, feat), jnp.float32),
            pltpu.SemaphoreType.DMA,
            pltpu.SemaphoreType.DMA,
            pltpu.SemaphoreType.DMA,
            pltpu.SemaphoreType.DMA,
            pltpu.SemaphoreType.DMA,
            pltpu.SemaphoreType.DMA,
        ],
    )
    def sc_kernel(x_hbm, idx_hbm, out_hbm,
                  idx0, idx1, rows0, rows1, agg,
                  isem0, isem1, gsem0, gsem1, ssem0, ssem1):
        c = lax.axis_index("c")
        s = lax.axis_index("s")
        wid = c * N_SUBCORES + s
        cpt = jnp.where(c == 0, cpt0, cpt1)
        idx_v = (idx0, idx1)
        rows_v = (rows0, rows1)
        isem = (isem0, isem1)
        gsem = (gsem0, gsem1)
        ssem = (ssem0, ssem1)

        def zero_row(r, carry):
            for j in range(n_fg):
                rows0[r, pl.ds(j * LANES, LANES)] = jnp.zeros(
                    (LANES,), jnp.float32)
            return carry
        lax.fori_loop(0, K_EDGES, zero_row, 0)

        base = s * rows_per_tile
        n_full = rows_per_tile // K_EDGES
        for t in range(n_full):
            pltpu.sync_copy(rows0, agg.at[pl.ds(base + t * K_EDGES, K_EDGES)])
        plsc.subcore_barrier()

        pltpu.sync_copy(idx_hbm.at[wid, 0], idx0)
        pltpu.async_copy(x_hbm.at[idx0.at[0]], rows0, gsem0)
        pltpu.async_copy(idx_hbm.at[wid, 1], idx1, isem1)

        def pair_body(k, carry):
            for u in range(2):
                i = 2 * k + u
                slot = u
                nslot = 1 - u
                pltpu.make_async_copy(
                    x_hbm.at[idx_v[slot].at[0]], rows_v[slot],
                    gsem[slot]).wait()

                @pl.when(jnp.logical_and(i + 1 < cpt, i >= 1))
                def _():
                    pltpu.make_async_copy(
                        rows_v[nslot], agg.at[idx_v[nslot].at[1]],
                        ssem[nslot]).wait()

                @pl.when(i + 1 < cpt)
                def _():
                    pltpu.make_async_copy(
                        idx_hbm.at[wid, i + 1], idx_v[nslot],
                        isem[nslot]).wait()
                    pltpu.async_copy(
                        x_hbm.at[idx_v[nslot].at[0]], rows_v[nslot],
                        gsem[nslot])

                def group_body(g, carry2):
                    wv = lax.bitcast_convert_type(
                        idx_v[slot][2, pl.ds(g * LANES, LANES)], jnp.float32)
                    for l in range(LANES):
                        wb = _lane_broadcast(wv, l)
                        for j in range(n_fg):
                            sl = pl.ds(j * LANES, LANES)
                            rows_v[slot][g * LANES + l, sl] = \
                                rows_v[slot][g * LANES + l, sl] * wb
                    return carry2
                lax.fori_loop(0, K_EDGES // LANES, group_body, 0)

                pltpu.async_copy(rows_v[slot], agg.at[idx_v[slot].at[1]],
                                 ssem[slot], add=True)

                @pl.when(i + 2 < cpt)
                def _():
                    pltpu.async_copy(
                        idx_hbm.at[wid, i + 2], idx_v[slot], isem[slot])
            return carry
        lax.fori_loop(0, cpt // 2, pair_body, 0)

        for slot in range(2):
            pltpu.make_async_copy(
                rows_v[slot], agg.at[idx_v[slot].at[1]],
                ssem[slot]).wait()

        plsc.subcore_barrier()
        pltpu.sync_copy(agg.at[pl.ds(base, rows_per_tile)],
                        out_hbm.at[c, pl.ds(base, rows_per_tile)])

    return sc_kernel(x, idx_r)


def _project(parts, W, b, m):
    feat = parts.shape[2]
    bm = 1000

    def mm_kernel(p_ref, w_ref, b_ref, o_ref):
        acc = p_ref[0] + p_ref[1]
        o_ref[...] = jnp.dot(acc, w_ref[...],
                             preferred_element_type=jnp.float32) \
            + b_ref[...][None, :]

    return pl.pallas_call(
        mm_kernel,
        grid=(m // bm,),
        in_specs=[
            pl.BlockSpec((N_CORES, bm, feat), lambda i: (0, i, 0)),
            pl.BlockSpec((feat, feat), lambda i: (0, 0)),
            pl.BlockSpec((feat,), lambda i: (0,)),
        ],
        out_specs=pl.BlockSpec((bm, feat), lambda i: (i, 0)),
        out_shape=jax.ShapeDtypeStruct((m, feat), jnp.float32),
    )(parts, W, b)


def kernel(x, edge_index, edge_weight, W, b):
    n_nodes = x.shape[0]
    n_edges = edge_weight.shape[0]
    cpt0, cpt1 = _split_chunks(n_edges)
    cpt_max = max(cpt0, cpt1)
    padded = N_SUBCORES * (cpt0 + cpt1) * K_EDGES
    pad = padded - n_edges

    dst = edge_index[0]
    src = edge_index[1]
    ew = edge_weight
    if pad:
        dst = jnp.concatenate([dst, jnp.zeros((pad,), dst.dtype)])
        src = jnp.concatenate([src, jnp.zeros((pad,), src.dtype)])
        ew = jnp.concatenate([ew, jnp.zeros((pad,), ew.dtype)])

    def to_tiles(a):
        n0 = N_SUBCORES * cpt0 * K_EDGES
        a0 = a[:n0].reshape(N_SUBCORES, cpt0, K_EDGES)
        a1 = a[n0:].reshape(N_SUBCORES, cpt1, K_EDGES)
        a0 = jnp.pad(a0, ((0, 0), (0, cpt_max - cpt0), (0, 0)))
        a1 = jnp.pad(a1, ((0, 0), (0, cpt_max - cpt1), (0, 0)))
        return jnp.concatenate([a0, a1], axis=0)

    idx_r = jnp.stack(
        [to_tiles(src),
         to_tiles(dst),
         to_tiles(lax.bitcast_convert_type(ew, jnp.int32))],
        axis=2)

    parts = _sc_aggregate(x, idx_r, cpt0, cpt1, n_nodes)
    return _project(parts, W, b, n_nodes)
